# baseline (speedup 1.0000x reference)
import functools

import jax
import jax.numpy as jnp
from jax import lax
from jax.experimental import pallas as pl
from jax.experimental.pallas import tpu as pltpu
from jax.experimental.pallas import tpu_sc as plsc

H_DIM = 128
L = 16
NC, NS = 2, 16
NW = NC * NS
B = 16384
B_PER_W = B // NW
CH = 64
NCH = B_PER_W // CH
NSLOT = 3
NJ = H_DIM // L
RB = 8

_mesh = plsc.VectorSubcoreMesh(core_axis_name="c", subcore_axis_name="s",
                               num_cores=NC, num_subcores=NS)


@functools.partial(
    pl.kernel,
    out_type=jax.ShapeDtypeStruct((B,), jnp.float32),
    mesh=_mesh,
    compiler_params=pltpu.CompilerParams(needs_layout_passes=False),
    scratch_types=[
        pltpu.VMEM((B_PER_W,), jnp.int32),
        pltpu.VMEM((B_PER_W,), jnp.int32),
        pltpu.VMEM((B_PER_W,), jnp.int32),
        pltpu.VMEM((NSLOT, CH, H_DIM), jnp.float32),
        pltpu.VMEM((NSLOT, CH, H_DIM), jnp.float32),
        pltpu.VMEM((NSLOT, CH, H_DIM), jnp.float32),
        pltpu.VMEM((B_PER_W + L,), jnp.float32),
        pltpu.SemaphoreType.DMA,
        pltpu.SemaphoreType.DMA,
        pltpu.SemaphoreType.DMA,
    ],
)
def _score_kernel(embs_hbm, wrel_hbm, hidx_hbm, ridx_hbm, tidx_hbm, out_hbm,
                  hidx_v, ridx_v, tidx_v, h_v, r_v, t_v, out_v,
                  sem0, sem1, sem2):
    wid = lax.axis_index("s") * NC + lax.axis_index("c")
    base = wid * B_PER_W
    pltpu.sync_copy(hidx_hbm.at[pl.ds(base, B_PER_W)], hidx_v)
    pltpu.sync_copy(ridx_hbm.at[pl.ds(base, B_PER_W)], ridx_v)
    pltpu.sync_copy(tidx_hbm.at[pl.ds(base, B_PER_W)], tidx_v)

    last_lane = lax.iota(jnp.int32, L) == (L - 1)
    sems = (sem0, sem1, sem2)

    def fire(c):
        s = c % NSLOT
        sem = sems[s]
        sl = pl.ds(c * CH, CH)
        return (
            pltpu.async_copy(embs_hbm.at[hidx_v.at[sl]], h_v.at[s], sem),
            pltpu.async_copy(wrel_hbm.at[ridx_v.at[sl]], r_v.at[s], sem),
            pltpu.async_copy(embs_hbm.at[tidx_v.at[sl]], t_v.at[s], sem),
        )

    pending = [fire(0), fire(1)]
    for c in range(NCH):
        for cp in pending.pop(0):
            cp.wait()
        if c + 2 < NCH:
            pending.append(fire(c + 2))
        s = c % NSLOT

        def row_blk(g, _, c=c, s=s):
            for rr in range(RB):
                b = g * RB + rr
                acc = (h_v[s, b, pl.ds(0, L)] * r_v[s, b, pl.ds(0, L)]
                       * t_v[s, b, pl.ds(0, L)])
                for j in range(1, NJ):
                    acc = acc + (h_v[s, b, pl.ds(j * L, L)]
                                 * r_v[s, b, pl.ds(j * L, L)]
                                 * t_v[s, b, pl.ds(j * L, L)])
                tot = lax.cumsum(acc, axis=0)
                plsc.store_compressed(out_v.at[pl.ds(c * CH + b, L)], tot,
                                      mask=last_lane)
            return 0

        lax.fori_loop(0, CH // RB, row_blk, 0, unroll=False)

    pltpu.sync_copy(out_v.at[pl.ds(0, B_PER_W)],
                    out_hbm.at[pl.ds(base, B_PER_W)])


def kernel(embs, sample, w_relation):
    sample = sample.astype(jnp.int32)
    out = _score_kernel(embs, w_relation, sample[0], sample[1], sample[2])
    return out.reshape(B, 1)


# speedup vs baseline: 1.0174x; 1.0174x over previous
import functools

import jax
import jax.numpy as jnp
from jax import lax
from jax.experimental import pallas as pl
from jax.experimental.pallas import tpu as pltpu
from jax.experimental.pallas import tpu_sc as plsc

H_DIM = 128
L = 16
NC, NS = 2, 16
NW = NC * NS
B = 16384
B_PER_W = B // NW
CH = 64
NCH = B_PER_W // CH
NSLOT = 4
NJ = H_DIM // L
RB = 4

_mesh = plsc.VectorSubcoreMesh(core_axis_name="c", subcore_axis_name="s",
                               num_cores=NC, num_subcores=NS)


@functools.partial(
    pl.kernel,
    out_type=jax.ShapeDtypeStruct((B,), jnp.float32),
    mesh=_mesh,
    compiler_params=pltpu.CompilerParams(needs_layout_passes=False),
    scratch_types=[
        pltpu.VMEM((B_PER_W,), jnp.int32),
        pltpu.VMEM((B_PER_W,), jnp.int32),
        pltpu.VMEM((B_PER_W,), jnp.int32),
        pltpu.VMEM((NSLOT, CH, H_DIM), jnp.float32),
        pltpu.VMEM((NSLOT, CH, H_DIM), jnp.float32),
        pltpu.VMEM((NSLOT, CH, H_DIM), jnp.float32),
        pltpu.VMEM((B_PER_W + L,), jnp.float32),
        pltpu.SemaphoreType.DMA,
        pltpu.SemaphoreType.DMA,
        pltpu.SemaphoreType.DMA,
        pltpu.SemaphoreType.DMA,
    ],
)
def _score_kernel(embs_hbm, wrel_hbm, hidx_hbm, ridx_hbm, tidx_hbm, out_hbm,
                  hidx_v, ridx_v, tidx_v, h_v, r_v, t_v, out_v,
                  sem0, sem1, sem2, sem3):
    wid = lax.axis_index("s") * NC + lax.axis_index("c")
    base = wid * B_PER_W
    pltpu.sync_copy(hidx_hbm.at[pl.ds(base, B_PER_W)], hidx_v)
    pltpu.sync_copy(ridx_hbm.at[pl.ds(base, B_PER_W)], ridx_v)
    pltpu.sync_copy(tidx_hbm.at[pl.ds(base, B_PER_W)], tidx_v)

    last_lane = lax.iota(jnp.int32, L) == (L - 1)
    sems = (sem0, sem1, sem2, sem3)

    def fire(c):
        s = c % NSLOT
        sem = sems[s]
        sl = pl.ds(c * CH, CH)
        return (
            pltpu.async_copy(embs_hbm.at[hidx_v.at[sl]], h_v.at[s], sem),
            pltpu.async_copy(wrel_hbm.at[ridx_v.at[sl]], r_v.at[s], sem),
            pltpu.async_copy(embs_hbm.at[tidx_v.at[sl]], t_v.at[s], sem),
        )

    pending = [fire(0), fire(1), fire(2)]
    for c in range(NCH):
        for cp in pending.pop(0):
            cp.wait()
        if c + 3 < NCH:
            pending.append(fire(c + 3))
        s = c % NSLOT

        def row_blk(g, _, c=c, s=s):
            for rr in range(RB):
                b = g * RB + rr
                acc = (h_v[s, b, pl.ds(0, L)] * r_v[s, b, pl.ds(0, L)]
                       * t_v[s, b, pl.ds(0, L)])
                for j in range(1, NJ):
                    acc = acc + (h_v[s, b, pl.ds(j * L, L)]
                                 * r_v[s, b, pl.ds(j * L, L)]
                                 * t_v[s, b, pl.ds(j * L, L)])
                tot = lax.cumsum(acc, axis=0)
                plsc.store_compressed(out_v.at[pl.ds(c * CH + b, L)], tot,
                                      mask=last_lane)
            return 0

        lax.fori_loop(0, CH // RB, row_blk, 0, unroll=False)

    pltpu.sync_copy(out_v.at[pl.ds(0, B_PER_W)],
                    out_hbm.at[pl.ds(base, B_PER_W)])


def kernel(embs, sample, w_relation):
    sample = sample.astype(jnp.int32)
    out = _score_kernel(embs, w_relation, sample[0], sample[1], sample[2])
    return out.reshape(B, 1)


# speedup vs baseline: 1.0366x; 1.0189x over previous
import functools

import jax
import jax.numpy as jnp
from jax import lax
from jax.experimental import pallas as pl
from jax.experimental.pallas import tpu as pltpu
from jax.experimental.pallas import tpu_sc as plsc

H_DIM = 128
L = 16
NC, NS = 2, 16
NW = NC * NS
B = 16384
B_PER_W = B // NW
CH = 64
NCH = B_PER_W // CH
NSLOT = 3
NJ = H_DIM // L
RB = 4

_mesh = plsc.VectorSubcoreMesh(core_axis_name="c", subcore_axis_name="s",
                               num_cores=NC, num_subcores=NS)


@functools.partial(
    pl.kernel,
    out_type=jax.ShapeDtypeStruct((B,), jnp.float32),
    mesh=_mesh,
    compiler_params=pltpu.CompilerParams(needs_layout_passes=False),
    scratch_types=[
        pltpu.VMEM((B_PER_W,), jnp.int32),
        pltpu.VMEM((B_PER_W,), jnp.int32),
        pltpu.VMEM((B_PER_W,), jnp.int32),
        pltpu.VMEM((NSLOT, CH, H_DIM), jnp.float32),
        pltpu.VMEM((NSLOT, CH, H_DIM), jnp.float32),
        pltpu.VMEM((NSLOT, CH, H_DIM), jnp.float32),
        pltpu.VMEM((B_PER_W + L,), jnp.float32),
        pltpu.VMEM_SHARED((1000, H_DIM), jnp.float32),
        pltpu.VMEM_SHARED((1000, H_DIM), jnp.float32),
        pltpu.SemaphoreType.DMA,
        pltpu.SemaphoreType.DMA,
        pltpu.SemaphoreType.DMA,
    ],
)
def _score_kernel(embs_hbm, wrel_hbm, hidx_hbm, ridx_hbm, tidx_hbm, out_hbm,
                  hidx_v, ridx_v, tidx_v, h_v, r_v, t_v, out_v,
                  shared_e, shared_w, sem0, sem1, sem2):
    sid = lax.axis_index("s")
    wid = sid * NC + lax.axis_index("c")
    base = wid * B_PER_W
    @pl.when(sid == 0)
    def _stage():
        pltpu.sync_copy(embs_hbm.at[pl.ds(0, 1000)], shared_e)
        pltpu.sync_copy(wrel_hbm, shared_w)

    pltpu.sync_copy(hidx_hbm.at[pl.ds(base, B_PER_W)], hidx_v)
    pltpu.sync_copy(ridx_hbm.at[pl.ds(base, B_PER_W)], ridx_v)
    pltpu.sync_copy(tidx_hbm.at[pl.ds(base, B_PER_W)], tidx_v)
    plsc.subcore_barrier()

    last_lane = lax.iota(jnp.int32, L) == (L - 1)
    sems = (sem0, sem1, sem2)

    def fire(c):
        s = c % NSLOT
        sem = sems[s]
        sl = pl.ds(c * CH, CH)
        return (
            pltpu.async_copy(shared_e.at[hidx_v.at[sl]], h_v.at[s], sem),
            pltpu.async_copy(shared_w.at[ridx_v.at[sl]], r_v.at[s], sem),
            pltpu.async_copy(shared_e.at[tidx_v.at[sl]], t_v.at[s], sem),
        )

    pending = [fire(0), fire(1)]
    for c in range(NCH):
        for cp in pending.pop(0):
            cp.wait()
        if c + 2 < NCH:
            pending.append(fire(c + 2))
        s = c % NSLOT

        def row_blk(g, _, c=c, s=s):
            for rr in range(RB):
                b = g * RB + rr
                acc = (h_v[s, b, pl.ds(0, L)] * r_v[s, b, pl.ds(0, L)]
                       * t_v[s, b, pl.ds(0, L)])
                for j in range(1, NJ):
                    acc = acc + (h_v[s, b, pl.ds(j * L, L)]
                                 * r_v[s, b, pl.ds(j * L, L)]
                                 * t_v[s, b, pl.ds(j * L, L)])
                tot = lax.cumsum(acc, axis=0)
                plsc.store_compressed(out_v.at[pl.ds(c * CH + b, L)], tot,
                                      mask=last_lane)
            return 0

        lax.fori_loop(0, CH // RB, row_blk, 0, unroll=False)

    pltpu.sync_copy(out_v.at[pl.ds(0, B_PER_W)],
                    out_hbm.at[pl.ds(base, B_PER_W)])


def kernel(embs, sample, w_relation):
    sample = sample.astype(jnp.int32)
    out = _score_kernel(embs, w_relation, sample[0], sample[1], sample[2])
    return out.reshape(B, 1)


# speedup vs baseline: 1.0837x; 1.0454x over previous
import functools

import jax
import jax.numpy as jnp
from jax import lax
from jax.experimental import pallas as pl
from jax.experimental.pallas import tpu as pltpu
from jax.experimental.pallas import tpu_sc as plsc

H_DIM = 128
L = 16
NC, NS = 2, 16
NW = NC * NS
B = 16384
B_PER_W = B // NW
CH = 64
NCH = B_PER_W // CH
NSLOT = 3
NJ = H_DIM // L
RB = 4

_mesh = plsc.VectorSubcoreMesh(core_axis_name="c", subcore_axis_name="s",
                               num_cores=NC, num_subcores=NS)


@functools.partial(
    pl.kernel,
    out_type=jax.ShapeDtypeStruct((B,), jnp.float32),
    mesh=_mesh,
    compiler_params=pltpu.CompilerParams(needs_layout_passes=False),
    scratch_types=[
        pltpu.VMEM((B_PER_W,), jnp.int32),
        pltpu.VMEM((B_PER_W,), jnp.int32),
        pltpu.VMEM((B_PER_W,), jnp.int32),
        pltpu.VMEM((NSLOT, CH, H_DIM), jnp.float32),
        pltpu.VMEM((NSLOT, CH, H_DIM), jnp.float32),
        pltpu.VMEM((NSLOT, CH, H_DIM), jnp.float32),
        pltpu.VMEM((B_PER_W + L,), jnp.float32),
        pltpu.SemaphoreType.DMA,
        pltpu.SemaphoreType.DMA,
        pltpu.SemaphoreType.DMA,
    ],
)
def _score_kernel(embs_hbm, wrel_hbm, hidx_hbm, ridx_hbm, tidx_hbm, out_hbm,
                  hidx_v, ridx_v, tidx_v, h_v, r_v, t_v, out_v,
                  sem0, sem1, sem2):
    wid = lax.axis_index("s") * NC + lax.axis_index("c")
    base = wid * B_PER_W
    icp = (
        pltpu.async_copy(hidx_hbm.at[pl.ds(base, B_PER_W)], hidx_v, sem0),
        pltpu.async_copy(ridx_hbm.at[pl.ds(base, B_PER_W)], ridx_v, sem0),
        pltpu.async_copy(tidx_hbm.at[pl.ds(base, B_PER_W)], tidx_v, sem0),
    )
    for cp in icp:
        cp.wait()

    last_lane = lax.iota(jnp.int32, L) == (L - 1)
    sems = (sem0, sem1, sem2)

    def fire(c):
        s = c % NSLOT
        sem = sems[s]
        sl = pl.ds(c * CH, CH)
        return (
            pltpu.async_copy(embs_hbm.at[hidx_v.at[sl]], h_v.at[s], sem),
            pltpu.async_copy(wrel_hbm.at[ridx_v.at[sl]], r_v.at[s], sem),
            pltpu.async_copy(embs_hbm.at[tidx_v.at[sl]], t_v.at[s], sem),
        )

    pending = [fire(0), fire(1)]
    for c in range(NCH):
        for cp in pending.pop(0):
            cp.wait()
        if c + 2 < NCH:
            pending.append(fire(c + 2))
        s = c % NSLOT

        def row_blk(g, _, c=c, s=s):
            for rr in range(RB):
                b = g * RB + rr
                acc = (h_v[s, b, pl.ds(0, L)] * r_v[s, b, pl.ds(0, L)]
                       * t_v[s, b, pl.ds(0, L)])
                for j in range(1, NJ):
                    acc = acc + (h_v[s, b, pl.ds(j * L, L)]
                                 * r_v[s, b, pl.ds(j * L, L)]
                                 * t_v[s, b, pl.ds(j * L, L)])
                tot = lax.cumsum(acc, axis=0)
                plsc.store_compressed(out_v.at[pl.ds(c * CH + b, L)], tot,
                                      mask=last_lane)
            return 0

        lax.fori_loop(0, CH // RB, row_blk, 0, unroll=False)

    pltpu.sync_copy(out_v.at[pl.ds(0, B_PER_W)],
                    out_hbm.at[pl.ds(base, B_PER_W)])


def kernel(embs, sample, w_relation):
    sample = sample.astype(jnp.int32)
    out = _score_kernel(embs, w_relation, sample[0], sample[1], sample[2])
    return out.reshape(B, 1)
